# baseline (speedup 1.0000x reference)
import functools

import jax
import jax.numpy as jnp
from jax import lax
from jax.experimental import pallas as pl
from jax.experimental.pallas import tpu as pltpu
from jax.experimental.pallas import tpu_sc as plsc

N = 10000
NPAD = 10240
E = 320000
EPAD = 327680
D = 128
G = 64
EPS = 1e-5

NC = 2
NS = 16
NW = NC * NS
CH = 128
NCHUNK = EPAD // CH
CPW = NCHUNK // NW
NH = 2
HC = CPW // NH
NCR = E // CH
NCP = NCHUNK - NCR
RPT = NPAD // NS

BR = 1024
NBLK = NPAD // BR


def _sc_mesh():
    return plsc.VectorSubcoreMesh(core_axis_name="c", subcore_axis_name="s",
                                  num_cores=NC, num_subcores=NS)


_SPLIT = (NW - 1) * CPW


def _load_idx(real_hbm, tail_hbm, cbase, buf):
    @pl.when(cbase < _SPLIT)
    def _():
        pltpu.sync_copy(real_hbm.at[pl.ds(cbase, HC)], buf)

    @pl.when(cbase >= _SPLIT)
    def _():
        pltpu.sync_copy(tail_hbm.at[pl.ds(cbase - _SPLIT, HC)], buf)


@functools.cache
def _build_sc_degree():
    return functools.partial(
        pl.kernel,
        out_type=jax.ShapeDtypeStruct((NC, NPAD), jnp.float32),
        mesh=_sc_mesh(),
        scratch_types=[
            pltpu.VMEM((CPW, CH), jnp.int32),
            pltpu.VMEM((CH,), jnp.float32),
            pltpu.VMEM((RPT,), jnp.float32),
            pltpu.VMEM_SHARED((NPAD,), jnp.float32),
        ],
    )(_sc_degree_body)


def _sc_degree(dst2, dtail):
    return _build_sc_degree()(dst2, dtail)


def _sc_degree_body(dst_hbm, tail_hbm, out_hbm, dst_v, ones_v, zeros_v, hist):
    c = lax.axis_index("c")
    s = lax.axis_index("s")
    wid = c * NS + s

    def fill_ones(i, carry):
        ones_v[pl.ds(i * 16, 16)] = jnp.ones((16,), jnp.float32)
        return carry

    lax.fori_loop(0, CH // 16, fill_ones, 0)

    def fill_zeros(i, carry):
        zeros_v[pl.ds(i * 16, 16)] = jnp.zeros((16,), jnp.float32)
        return carry

    lax.fori_loop(0, RPT // 16, fill_zeros, 0)
    pltpu.sync_copy(zeros_v, hist.at[pl.ds(s * RPT, RPT)])
    _load_idx(dst_hbm, tail_hbm, wid * CPW, dst_v.at[pl.ds(0, HC)])
    _load_idx(dst_hbm, tail_hbm, wid * CPW + HC, dst_v.at[pl.ds(HC, HC)])
    plsc.subcore_barrier()

    def body(j, carry):
        pltpu.sync_copy(ones_v, hist.at[dst_v.at[j]], add=True)
        return carry

    lax.fori_loop(0, CPW, body, 0)
    plsc.subcore_barrier()
    pltpu.sync_copy(hist.at[pl.ds(s * RPT, RPT)], out_hbm.at[c, pl.ds(s * RPT, RPT)])


@functools.cache
def _build_sc_agg():
    return functools.partial(
        pl.kernel,
        out_type=jax.ShapeDtypeStruct((NC, NPAD, D), jnp.float32),
        mesh=_sc_mesh(),
        scratch_types=[
            pltpu.VMEM((HC, CH), jnp.int32),
            pltpu.VMEM((HC, CH), jnp.int32),
            pltpu.VMEM((2, CH, D), jnp.float32),
            pltpu.VMEM_SHARED((NPAD, D), jnp.float32),
            pltpu.SemaphoreType.DMA,
            pltpu.SemaphoreType.DMA,
        ],
    )(_sc_agg_body)


def _sc_agg(hp, src2, dst2, stail, dtail):
    return _build_sc_agg()(hp, src2, dst2, stail, dtail)


def _sc_agg_body(hp_hbm, src_hbm, dst_hbm, stail_hbm, dtail_hbm, out_hbm,
                 src_v, dst_v, rows_v, acc, gsem, ssem):
    c = lax.axis_index("c")
    s = lax.axis_index("s")
    wid = c * NS + s
    rbase = s * RPT
    @pl.when(c == 0)
    def _():
        pltpu.sync_copy(hp_hbm.at[pl.ds(rbase, RPT)], acc.at[pl.ds(rbase, RPT)])

    @pl.when(c == 1)
    def _():
        def zfill(i, carry):
            rows_v[0, i // 8, pl.ds((i % 8) * 16, 16)] = jnp.zeros((16,), jnp.float32)
            return carry

        lax.fori_loop(0, CH * 8, zfill, 0)

        def zcopy(i, carry):
            pltpu.sync_copy(rows_v.at[0], acc.at[pl.ds(rbase + i * CH, CH)])
            return carry

        lax.fori_loop(0, RPT // CH, zcopy, 0)

    plsc.subcore_barrier()

    def stage(h, carry):
        cbase = wid * CPW + h * HC
        _load_idx(src_hbm, stail_hbm, cbase, src_v)
        _load_idx(dst_hbm, dtail_hbm, cbase, dst_v)
        pltpu.async_copy(hp_hbm.at[src_v.at[0]], rows_v.at[0], gsem)
        pltpu.async_copy(hp_hbm.at[src_v.at[1]], rows_v.at[1], gsem)

        def body(i, c2):
            j = i * 2
            pltpu.make_async_copy(hp_hbm.at[src_v.at[j]], rows_v.at[0], gsem).wait()
            pltpu.make_async_copy(rows_v.at[0], acc.at[dst_v.at[j]], ssem).start(add=True)
            pltpu.make_async_copy(hp_hbm.at[src_v.at[j + 1]], rows_v.at[1], gsem).wait()
            pltpu.make_async_copy(rows_v.at[1], acc.at[dst_v.at[j + 1]], ssem).start(add=True)
            pltpu.make_async_copy(rows_v.at[0], acc.at[dst_v.at[j]], ssem).wait()

            @pl.when(j + 2 < HC)
            def _():
                pltpu.async_copy(hp_hbm.at[src_v.at[j + 2]], rows_v.at[0], gsem)

            pltpu.make_async_copy(rows_v.at[1], acc.at[dst_v.at[j + 1]], ssem).wait()

            @pl.when(j + 3 < HC)
            def _():
                pltpu.async_copy(hp_hbm.at[src_v.at[j + 3]], rows_v.at[1], gsem)

            return c2

        lax.fori_loop(0, HC // 2, body, 0)
        return carry

    lax.fori_loop(0, NH, stage, 0)
    plsc.subcore_barrier()
    pltpu.sync_copy(acc.at[pl.ds(rbase, RPT)], out_hbm.at[c, pl.ds(rbase, RPT)])


def _dot(a, b):
    return jnp.dot(a, b, preferred_element_type=jnp.float32)


def _tc_pre(x, W0, degB):
    def body(x_ref, w_ref, deg_ref, o_ref):
        dinv = lax.rsqrt(deg_ref[...][:, 0:1])
        h = _dot(x_ref[...], w_ref[...]) * dinv
        row = pl.program_id(0) * BR + lax.broadcasted_iota(jnp.int32, (BR, D), 0)
        o_ref[...] = jnp.where(row < N, h, 0.0)

    return pl.pallas_call(
        body,
        grid=(NBLK,),
        in_specs=[pl.BlockSpec((BR, D), lambda i: (i, 0)),
                  pl.BlockSpec((D, D), lambda i: (0, 0)),
                  pl.BlockSpec((BR, 8), lambda i: (i, 0))],
        out_specs=pl.BlockSpec((BR, D), lambda i: (i, 0)),
        out_shape=jax.ShapeDtypeStruct((NPAD, D), jnp.float32),
    )(x, W0, degB)


def _tc_mid(p, degB, b0r, g0r, be0r, W1):
    def body(p_ref, deg_ref, b_ref, g_ref, be_ref, w_ref, o_ref):
        dinv = lax.rsqrt(deg_ref[...][:, 0:1])
        u = dinv * (p_ref[0] + p_ref[1]) + b_ref[...]
        mu = jnp.mean(u, axis=-1, keepdims=True)
        var = jnp.mean((u - mu) ** 2, axis=-1, keepdims=True)
        t = (u - mu) * lax.rsqrt(var + EPS) * g_ref[...] + be_ref[...]
        r = jnp.maximum(t, 0.0)
        h2 = _dot(r, w_ref[...]) * dinv
        row = pl.program_id(0) * BR + lax.broadcasted_iota(jnp.int32, (BR, D), 0)
        o_ref[...] = jnp.where(row < N, h2, 0.0)

    blk = pl.BlockSpec((BR, D), lambda i: (i, 0))
    pblk = pl.BlockSpec((2, BR, D), lambda i: (0, i, 0))
    nblk8 = pl.BlockSpec((BR, 8), lambda i: (i, 0))
    one = pl.BlockSpec((1, D), lambda i: (0, 0))
    wspec = pl.BlockSpec((D, D), lambda i: (0, 0))
    return pl.pallas_call(
        body,
        grid=(NBLK,),
        in_specs=[pblk, nblk8, one, one, one, wspec],
        out_specs=blk,
        out_shape=jax.ShapeDtypeStruct((NPAD, D), jnp.float32),
    )(p, degB, b0r, g0r, be0r, W1)


def _tc_post(q, degB, b1r, g1r, be1r, batchB, Wh1, bh1r, Wh2p, bh2r):
    def body(q_ref, deg_ref, b_ref, g_ref, be_ref, bat_ref,
             wh1_ref, bh1_ref, wh2_ref, bh2_ref, o_ref, acc_ref):
        i = pl.program_id(0)

        @pl.when(i == 0)
        def _():
            acc_ref[...] = jnp.zeros_like(acc_ref)

        dinv = lax.rsqrt(deg_ref[...][:, 0:1])
        u = dinv * (q_ref[0] + q_ref[1]) + b_ref[...]
        mu = jnp.mean(u, axis=-1, keepdims=True)
        var = jnp.mean((u - mu) ** 2, axis=-1, keepdims=True)
        t = (u - mu) * lax.rsqrt(var + EPS) * g_ref[...] + be_ref[...]
        r = jnp.maximum(t, 0.0)
        row = i * BR + lax.broadcasted_iota(jnp.int32, (BR, G), 0)
        oh = ((bat_ref[...][:, 0:1] ==
               lax.broadcasted_iota(jnp.int32, (BR, G), 1)) & (row < N)
              ).astype(jnp.float32)
        lane = lax.broadcasted_iota(jnp.int32, (BR, D), 1)
        ones_col = jnp.where(lane == 0, 1.0, 0.0)
        re = jnp.concatenate([r, ones_col], axis=1)
        s = lax.dot_general(oh, re, (((0,), (0,)), ((), ())),
                            precision=lax.Precision.HIGHEST,
                            preferred_element_type=jnp.float32)
        acc_ref[...] += s

        @pl.when(i == NBLK - 1)
        def _():
            counts = acc_ref[:, D:D + 1]
            Z = acc_ref[:, :D] / jnp.maximum(counts, 1.0)
            hid = jnp.maximum(_dot(Z, wh1_ref[...]) + bh1_ref[...], 0.0)
            o_ref[...] = _dot(hid, wh2_ref[...]) + bh2_ref[...]

    blk = pl.BlockSpec((BR, D), lambda i: (i, 0))
    pblk = pl.BlockSpec((2, BR, D), lambda i: (0, i, 0))
    nblk8 = pl.BlockSpec((BR, 8), lambda i: (i, 0))
    one = pl.BlockSpec((1, D), lambda i: (0, 0))
    wspec = pl.BlockSpec((D, D), lambda i: (0, 0))
    return pl.pallas_call(
        body,
        grid=(NBLK,),
        in_specs=[pblk, nblk8, one, one, one, nblk8, wspec, one, wspec, one],
        out_specs=pl.BlockSpec((G, D), lambda i: (0, 0)),
        out_shape=jax.ShapeDtypeStruct((G, D), jnp.float32),
        scratch_shapes=[pltpu.VMEM((G, 2 * D), jnp.float32)],
    )(q, degB, b1r, g1r, be1r, batchB, Wh1, bh1r, Wh2p, bh2r)


def kernel(x, edge_index, batch, W0, b0, g0, be0, W1, b1, g1, be1, Wh1, bh1, Wh2, bh2):
    nreal = _SPLIT * CH
    src2 = edge_index[0, :nreal].reshape(_SPLIT, CH)
    dst2 = edge_index[1, :nreal].reshape(_SPLIT, CH)
    pad_e = N + jnp.arange(EPAD - E, dtype=jnp.int32) % (NPAD - N)
    stail = jnp.concatenate([edge_index[0, nreal:], pad_e]).reshape(CPW, CH)
    dtail = jnp.concatenate([edge_index[1, nreal:], pad_e]).reshape(CPW, CH)
    batchB = jnp.broadcast_to(batch[:, None], (N, 8))

    degp = _sc_degree(dst2, dtail)
    deg = degp[0] + degp[1] + 1.0
    degB = jnp.broadcast_to(deg[:, None], (NPAD, 8))

    b0r, g0r, be0r = b0.reshape(1, D), g0.reshape(1, D), be0.reshape(1, D)
    b1r, g1r, be1r = b1.reshape(1, D), g1.reshape(1, D), be1.reshape(1, D)
    bh1r = bh1.reshape(1, D)
    Wh2p = jnp.pad(Wh2, ((0, 0), (0, D - 1)))
    bh2r = jnp.pad(bh2, (0, D - 1)).reshape(1, D)

    hp1 = _tc_pre(x, W0, degB)
    p = _sc_agg(hp1, src2, dst2, stail, dtail)
    hp2 = _tc_mid(p, degB, b0r, g0r, be0r, W1)
    q = _sc_agg(hp2, src2, dst2, stail, dtail)
    logits_full = _tc_post(q, degB, b1r, g1r, be1r, batchB,
                           Wh1, bh1r, Wh2p, bh2r)
    return logits_full[:, :1]


# speedup vs baseline: 1.0174x; 1.0174x over previous
import functools

import jax
import jax.numpy as jnp
from jax import lax
from jax.experimental import pallas as pl
from jax.experimental.pallas import tpu as pltpu
from jax.experimental.pallas import tpu_sc as plsc

N = 10000
NPAD = 10240
E = 320000
EPAD = 327680
D = 128
G = 64
EPS = 1e-5

NC = 2
NS = 16
NW = NC * NS
CH = 128
NCHUNK = EPAD // CH
CPW = NCHUNK // NW
NH = 2
HC = CPW // NH
NCR = E // CH
NCP = NCHUNK - NCR
RPT = NPAD // NS

BR = 1024
NBLK = NPAD // BR


def _sc_mesh():
    return plsc.VectorSubcoreMesh(core_axis_name="c", subcore_axis_name="s",
                                  num_cores=NC, num_subcores=NS)


_SPLIT = (NW - 1) * CPW


def _load_idx(real_hbm, tail_hbm, cbase, buf):
    @pl.when(cbase < _SPLIT)
    def _():
        pltpu.sync_copy(real_hbm.at[pl.ds(cbase, HC)], buf)

    @pl.when(cbase >= _SPLIT)
    def _():
        pltpu.sync_copy(tail_hbm.at[pl.ds(cbase - _SPLIT, HC)], buf)


@functools.cache
def _build_sc_degree():
    return functools.partial(
        pl.kernel,
        out_type=jax.ShapeDtypeStruct((NC, NPAD), jnp.float32),
        mesh=_sc_mesh(),
        scratch_types=[
            pltpu.VMEM((CPW, CH), jnp.int32),
            pltpu.VMEM((CH,), jnp.float32),
            pltpu.VMEM((RPT,), jnp.float32),
            pltpu.VMEM_SHARED((NPAD,), jnp.float32),
        ],
    )(_sc_degree_body)


def _sc_degree(dst2, dtail):
    return _build_sc_degree()(dst2, dtail)


def _sc_degree_body(dst_hbm, tail_hbm, out_hbm, dst_v, ones_v, zeros_v, hist):
    c = lax.axis_index("c")
    s = lax.axis_index("s")
    wid = c * NS + s

    def fill_ones(i, carry):
        ones_v[pl.ds(i * 16, 16)] = jnp.ones((16,), jnp.float32)
        return carry

    lax.fori_loop(0, CH // 16, fill_ones, 0)

    def fill_zeros(i, carry):
        zeros_v[pl.ds(i * 16, 16)] = jnp.zeros((16,), jnp.float32)
        return carry

    lax.fori_loop(0, RPT // 16, fill_zeros, 0)
    pltpu.sync_copy(zeros_v, hist.at[pl.ds(s * RPT, RPT)])
    _load_idx(dst_hbm, tail_hbm, wid * CPW, dst_v.at[pl.ds(0, HC)])
    _load_idx(dst_hbm, tail_hbm, wid * CPW + HC, dst_v.at[pl.ds(HC, HC)])
    plsc.subcore_barrier()

    def body(j, carry):
        pltpu.sync_copy(ones_v, hist.at[dst_v.at[j]], add=True)
        return carry

    lax.fori_loop(0, CPW, body, 0)
    plsc.subcore_barrier()
    pltpu.sync_copy(hist.at[pl.ds(s * RPT, RPT)], out_hbm.at[c, pl.ds(s * RPT, RPT)])


@functools.cache
def _build_sc_agg():
    return functools.partial(
        pl.kernel,
        out_type=jax.ShapeDtypeStruct((NC, NPAD, D), jnp.float32),
        mesh=_sc_mesh(),
        scratch_types=[
            pltpu.VMEM((HC, CH), jnp.int32),
            pltpu.VMEM((HC, CH), jnp.int32),
            pltpu.VMEM((2, CH, D), jnp.float32),
            pltpu.VMEM_SHARED((NPAD, D), jnp.float32),
            pltpu.SemaphoreType.DMA,
            pltpu.SemaphoreType.DMA,
        ],
    )(_sc_agg_body)


def _sc_agg(hp, src2, dst2, stail, dtail):
    return _build_sc_agg()(hp, src2, dst2, stail, dtail)


def _sc_agg_body(hp_hbm, src_hbm, dst_hbm, stail_hbm, dtail_hbm, out_hbm,
                 src_v, dst_v, rows_v, acc, gsem, ssem):
    c = lax.axis_index("c")
    s = lax.axis_index("s")
    wid = c * NS + s
    rbase = s * RPT
    @pl.when(c == 0)
    def _():
        pltpu.sync_copy(hp_hbm.at[pl.ds(rbase, RPT)], acc.at[pl.ds(rbase, RPT)])

    @pl.when(c == 1)
    def _():
        def zfill(i, carry):
            rows_v[0, i // 8, pl.ds((i % 8) * 16, 16)] = jnp.zeros((16,), jnp.float32)
            return carry

        lax.fori_loop(0, CH * 8, zfill, 0)

        def zcopy(i, carry):
            pltpu.sync_copy(rows_v.at[0], acc.at[pl.ds(rbase + i * CH, CH)])
            return carry

        lax.fori_loop(0, RPT // CH, zcopy, 0)

    plsc.subcore_barrier()

    def stage(h, carry):
        cbase = wid * CPW + h * HC
        _load_idx(src_hbm, stail_hbm, cbase, src_v)
        _load_idx(dst_hbm, dtail_hbm, cbase, dst_v)
        pltpu.async_copy(hp_hbm.at[src_v.at[0]], rows_v.at[0], gsem)
        pltpu.async_copy(hp_hbm.at[src_v.at[1]], rows_v.at[1], gsem)

        def body(i, c2):
            j = i * 2
            pltpu.make_async_copy(hp_hbm.at[src_v.at[j]], rows_v.at[0], gsem).wait()
            pltpu.make_async_copy(rows_v.at[0], acc.at[dst_v.at[j]], ssem).start(add=True)
            pltpu.make_async_copy(hp_hbm.at[src_v.at[j + 1]], rows_v.at[1], gsem).wait()
            pltpu.make_async_copy(rows_v.at[1], acc.at[dst_v.at[j + 1]], ssem).start(add=True)
            pltpu.make_async_copy(rows_v.at[0], acc.at[dst_v.at[j]], ssem).wait()

            @pl.when(j + 2 < HC)
            def _():
                pltpu.async_copy(hp_hbm.at[src_v.at[j + 2]], rows_v.at[0], gsem)

            pltpu.make_async_copy(rows_v.at[1], acc.at[dst_v.at[j + 1]], ssem).wait()

            @pl.when(j + 3 < HC)
            def _():
                pltpu.async_copy(hp_hbm.at[src_v.at[j + 3]], rows_v.at[1], gsem)

            return c2

        lax.fori_loop(0, HC // 2, body, 0)
        return carry

    lax.fori_loop(0, NH, stage, 0)
    plsc.subcore_barrier()
    pltpu.sync_copy(acc.at[pl.ds(rbase, RPT)], out_hbm.at[c, pl.ds(rbase, RPT)])


def _dot(a, b):
    return jnp.dot(a, b, preferred_element_type=jnp.float32)


def _tc_pre(x, W0, degB):
    def body(x_ref, w_ref, deg_ref, o_ref):
        dinv = lax.rsqrt(deg_ref[...].astype(jnp.float32))
        h = _dot(x_ref[...], w_ref[...]) * dinv
        row = pl.program_id(0) * BR + lax.broadcasted_iota(jnp.int32, (BR, D), 0)
        o_ref[...] = jnp.where(row < N, h, 0.0)

    return pl.pallas_call(
        body,
        grid=(NBLK,),
        in_specs=[pl.BlockSpec((BR, D), lambda i: (i, 0)),
                  pl.BlockSpec((D, D), lambda i: (0, 0)),
                  pl.BlockSpec((BR, D), lambda i: (i, 0))],
        out_specs=pl.BlockSpec((BR, D), lambda i: (i, 0)),
        out_shape=jax.ShapeDtypeStruct((NPAD, D), jnp.float32),
    )(x, W0, degB)


def _tc_mid(p, degB, b0r, g0r, be0r, W1):
    def body(p_ref, deg_ref, b_ref, g_ref, be_ref, w_ref, o_ref):
        dinv = lax.rsqrt(deg_ref[...].astype(jnp.float32))
        u = dinv * (p_ref[0] + p_ref[1]) + b_ref[...]
        mu = jnp.mean(u, axis=-1, keepdims=True)
        var = jnp.mean((u - mu) ** 2, axis=-1, keepdims=True)
        t = (u - mu) * lax.rsqrt(var + EPS) * g_ref[...] + be_ref[...]
        r = jnp.maximum(t, 0.0)
        h2 = _dot(r, w_ref[...]) * dinv
        row = pl.program_id(0) * BR + lax.broadcasted_iota(jnp.int32, (BR, D), 0)
        o_ref[...] = jnp.where(row < N, h2, 0.0)

    blk = pl.BlockSpec((BR, D), lambda i: (i, 0))
    pblk = pl.BlockSpec((2, BR, D), lambda i: (0, i, 0))
    one = pl.BlockSpec((1, D), lambda i: (0, 0))
    wspec = pl.BlockSpec((D, D), lambda i: (0, 0))
    return pl.pallas_call(
        body,
        grid=(NBLK,),
        in_specs=[pblk, blk, one, one, one, wspec],
        out_specs=blk,
        out_shape=jax.ShapeDtypeStruct((NPAD, D), jnp.float32),
    )(p, degB, b0r, g0r, be0r, W1)


def _tc_post(q, degB, b1r, g1r, be1r, batchB, Wh1, bh1r, Wh2p, bh2r):
    def body(q_ref, deg_ref, b_ref, g_ref, be_ref, bat_ref,
             wh1_ref, bh1_ref, wh2_ref, bh2_ref, o_ref, acc_ref):
        i = pl.program_id(0)

        @pl.when(i == 0)
        def _():
            acc_ref[...] = jnp.zeros_like(acc_ref)

        dinv = lax.rsqrt(deg_ref[...].astype(jnp.float32))
        u = dinv * (q_ref[0] + q_ref[1]) + b_ref[...]
        mu = jnp.mean(u, axis=-1, keepdims=True)
        var = jnp.mean((u - mu) ** 2, axis=-1, keepdims=True)
        t = (u - mu) * lax.rsqrt(var + EPS) * g_ref[...] + be_ref[...]
        r = jnp.maximum(t, 0.0)
        row = i * BR + lax.broadcasted_iota(jnp.int32, (BR, G), 0)
        oh = ((bat_ref[...][:, :G].astype(jnp.int32) ==
               lax.broadcasted_iota(jnp.int32, (BR, G), 1)) & (row < N)
              ).astype(jnp.float32)
        lane = lax.broadcasted_iota(jnp.int32, (BR, D), 1)
        ones_col = jnp.where(lane == 0, 1.0, 0.0)
        re = jnp.concatenate([r, ones_col], axis=1)
        s = lax.dot_general(oh, re, (((0,), (0,)), ((), ())),
                            precision=lax.Precision.HIGHEST,
                            preferred_element_type=jnp.float32)
        acc_ref[...] += s

        @pl.when(i == NBLK - 1)
        def _():
            counts = acc_ref[:, D:D + 1]
            Z = acc_ref[:, :D] / jnp.maximum(counts, 1.0)
            hid = jnp.maximum(_dot(Z, wh1_ref[...]) + bh1_ref[...], 0.0)
            o_ref[...] = _dot(hid, wh2_ref[...]) + bh2_ref[...]

    blk = pl.BlockSpec((BR, D), lambda i: (i, 0))
    pblk = pl.BlockSpec((2, BR, D), lambda i: (0, i, 0))
    one = pl.BlockSpec((1, D), lambda i: (0, 0))
    wspec = pl.BlockSpec((D, D), lambda i: (0, 0))
    return pl.pallas_call(
        body,
        grid=(NBLK,),
        in_specs=[pblk, blk, one, one, one, blk, wspec, one, wspec, one],
        out_specs=pl.BlockSpec((G, D), lambda i: (0, 0)),
        out_shape=jax.ShapeDtypeStruct((G, D), jnp.float32),
        scratch_shapes=[pltpu.VMEM((G, 2 * D), jnp.float32)],
    )(q, degB, b1r, g1r, be1r, batchB, Wh1, bh1r, Wh2p, bh2r)


def kernel(x, edge_index, batch, W0, b0, g0, be0, W1, b1, g1, be1, Wh1, bh1, Wh2, bh2):
    nreal = _SPLIT * CH
    src2 = edge_index[0, :nreal].reshape(_SPLIT, CH)
    dst2 = edge_index[1, :nreal].reshape(_SPLIT, CH)
    pad_e = N + jnp.arange(EPAD - E, dtype=jnp.int32) % (NPAD - N)
    stail = jnp.concatenate([edge_index[0, nreal:], pad_e]).reshape(CPW, CH)
    dtail = jnp.concatenate([edge_index[1, nreal:], pad_e]).reshape(CPW, CH)
    batchB = jnp.broadcast_to(batch[:, None], (N, D)).astype(jnp.int8)

    degp = _sc_degree(dst2, dtail)
    deg = degp[0] + degp[1] + 1.0
    degB = jnp.broadcast_to(deg[:, None], (NPAD, D)).astype(jnp.bfloat16)

    b0r, g0r, be0r = b0.reshape(1, D), g0.reshape(1, D), be0.reshape(1, D)
    b1r, g1r, be1r = b1.reshape(1, D), g1.reshape(1, D), be1.reshape(1, D)
    bh1r = bh1.reshape(1, D)
    Wh2p = jnp.pad(Wh2, ((0, 0), (0, D - 1)))
    bh2r = jnp.pad(bh2, (0, D - 1)).reshape(1, D)

    hp1 = _tc_pre(x, W0, degB)
    p = _sc_agg(hp1, src2, dst2, stail, dtail)
    hp2 = _tc_mid(p, degB, b0r, g0r, be0r, W1)
    q = _sc_agg(hp2, src2, dst2, stail, dtail)
    logits_full = _tc_post(q, degB, b1r, g1r, be1r, batchB,
                           Wh1, bh1r, Wh2p, bh2r)
    return logits_full[:, :1]
